# baseline (speedup 1.0000x reference)
import functools

import jax
import jax.numpy as jnp
from jax import lax
from jax.experimental import pallas as pl
from jax.experimental.pallas import tpu as pltpu
from jax.experimental.pallas import tpu_sc as plsc

BATCH = 16384
N_TYPES = 1000
TAB_PAD = 1024
NC, NS, L = 2, 16, 16
NW = NC * NS
CHUNK = BATCH // NW
SLICE = TAB_PAD // NS
LAST_BASE = N_TYPES - SLICE

_LN2 = 0.6931471805599453
_SQRT2 = 1.4142135623730951
_P = (
    1.0000083662233654,
    -0.4998235247953396,
    0.3325310803861114,
    -0.2552293761746607,
    0.2203877740075587,
    -0.1376623938583184,
)


def _log16(x):
    bits = plsc.bitcast(x, jnp.int32)
    e = (bits >> 23) - 127
    m = plsc.bitcast((bits & 0x007FFFFF) | 0x3F800000, jnp.float32)
    big = m > _SQRT2
    m = jnp.where(big, m * 0.5, m)
    e = e + jnp.where(big, 1, 0)
    t = m - 1.0
    p = _P[-1]
    for coef in _P[-2::-1]:
        p = p * t + coef
    return e.astype(jnp.float32) * _LN2 + p * t


_mesh = plsc.VectorSubcoreMesh(core_axis_name="c", subcore_axis_name="s")


@functools.partial(
    pl.kernel,
    mesh=_mesh,
    out_type=jax.ShapeDtypeStruct((BATCH,), jnp.float32),
    scratch_types=[
        pltpu.VMEM((SLICE,), jnp.float32),
        pltpu.VMEM((CHUNK,), jnp.int32),
        pltpu.VMEM((CHUNK,), jnp.float32),
        pltpu.VMEM_SHARED((TAB_PAD,), jnp.float32),
        pltpu.SemaphoreType.DMA,
        pltpu.SemaphoreType.DMA,
        pltpu.SemaphoreType.DMA,
    ],
    compiler_params=pltpu.CompilerParams(
        needs_layout_passes=False,
        skip_device_barrier=True,
        disable_bounds_checks=True,
        disable_semaphore_checks=True,
    ),
)
def _logprob_sc(c_hbm, tab_hbm, out_hbm, slice_v, idx_v, out_v, tab_sh,
                sem_t, sem_i, sem_g):
    sid = lax.axis_index("s")
    wid = sid * NC + lax.axis_index("c")
    base = wid * CHUNK
    tbase = jnp.minimum(sid * SLICE, LAST_BASE)
    tab_cp = pltpu.async_copy(tab_hbm.at[pl.ds(tbase, SLICE)], slice_v, sem_t)
    idx_cp = pltpu.async_copy(c_hbm.at[pl.ds(base, CHUNK)], idx_v, sem_i)
    tab_cp.wait()
    for j in range(SLICE // L):
        sl = pl.ds(j * L, L)
        slice_v[sl] = _log16(slice_v[sl])
    pltpu.sync_copy(slice_v, tab_sh.at[pl.ds(tbase, SLICE)])
    plsc.subcore_barrier()
    idx_cp.wait()
    pltpu.async_copy(tab_sh.at[idx_v], out_v, sem_g).wait()
    pltpu.sync_copy(out_v, out_hbm.at[pl.ds(base, CHUNK)])


def kernel(c, probabilities):
    return _logprob_sc(c.astype(jnp.int32), probabilities)


# speedup vs baseline: 1.0065x; 1.0065x over previous
import functools

import jax
import jax.numpy as jnp
from jax import lax
from jax.experimental import pallas as pl
from jax.experimental.pallas import tpu as pltpu
from jax.experimental.pallas import tpu_sc as plsc

BATCH = 16384
N_TYPES = 1000
TAB_PAD = 1024
NC, NS, L = 2, 16, 16
NW = NC * NS
CHUNK = BATCH // NW
SLICE = TAB_PAD // NS
LAST_BASE = N_TYPES - SLICE

_LN2 = 0.6931471805599453
_SQRT2 = 1.4142135623730951
_P = (
    1.0000083662233654,
    -0.4998235247953396,
    0.3325310803861114,
    -0.2552293761746607,
    0.2203877740075587,
    -0.1376623938583184,
)


def _log16(x):
    bits = plsc.bitcast(x, jnp.int32)
    e = (bits >> 23) - 127
    m = plsc.bitcast((bits & 0x007FFFFF) | 0x3F800000, jnp.float32)
    big = m > _SQRT2
    m = jnp.where(big, m * 0.5, m)
    e = e + jnp.where(big, 1, 0)
    t = m - 1.0
    p = _P[-1]
    for coef in _P[-2::-1]:
        p = p * t + coef
    return e.astype(jnp.float32) * _LN2 + p * t


_mesh = plsc.VectorSubcoreMesh(core_axis_name="c", subcore_axis_name="s")


@functools.partial(
    pl.kernel,
    mesh=_mesh,
    out_type=jax.ShapeDtypeStruct((BATCH,), jnp.float32),
    scratch_types=[
        pltpu.VMEM((SLICE,), jnp.float32),
        pltpu.VMEM((CHUNK,), jnp.int32),
        pltpu.VMEM((CHUNK,), jnp.float32),
        pltpu.VMEM_SHARED((TAB_PAD,), jnp.float32),
        pltpu.SemaphoreType.DMA,
        pltpu.SemaphoreType.DMA,
        pltpu.SemaphoreType.DMA,
    ],
    compiler_params=pltpu.CompilerParams(needs_layout_passes=False),
)
def _logprob_sc(c_hbm, tab_hbm, out_hbm, slice_v, idx_v, out_v, tab_sh,
                sem_t, sem_i, sem_g):
    sid = lax.axis_index("s")
    wid = sid * NC + lax.axis_index("c")
    base = wid * CHUNK
    tbase = jnp.minimum(sid * SLICE, LAST_BASE)
    tab_cp = pltpu.async_copy(tab_hbm.at[pl.ds(tbase, SLICE)], slice_v, sem_t)
    idx_cp = pltpu.async_copy(c_hbm.at[pl.ds(base, CHUNK)], idx_v, sem_i)
    tab_cp.wait()
    for j in range(SLICE // L):
        sl = pl.ds(j * L, L)
        slice_v[sl] = _log16(slice_v[sl])
    pltpu.sync_copy(slice_v, tab_sh.at[pl.ds(tbase, SLICE)])
    plsc.subcore_barrier()
    idx_cp.wait()
    pltpu.async_copy(tab_sh.at[idx_v], out_v, sem_g).wait()
    pltpu.sync_copy(out_v, out_hbm.at[pl.ds(base, CHUNK)])


def kernel(c, probabilities):
    return _logprob_sc(c.astype(jnp.int32), probabilities)
